# Initial kernel scaffold; baseline (speedup 1.0000x reference)
import jax
import jax.numpy as jnp
from jax.experimental import pallas as pl


def kernel(x, edge_index, params):
    raise NotImplementedError("write your pallas kernel here")


# speedup vs baseline: 9.2569x; 9.2569x over previous
import functools

import jax
import jax.numpy as jnp
from jax import lax
from jax.experimental import pallas as pl
from jax.experimental.pallas import tpu as pltpu
from jax.experimental.pallas import tpu_sc as plsc

N = 10000
NP = 10240
D = 128
LANE = 16
NTILES = 32
FPT = D // NTILES
CHUNK = 2000


def _pre_body(xT_ref, w_ref, asv_ref, adv_ref,
              hT_ref, as_ref, ad_ref, mx_ref, exs_ref):
    xT = xT_ref[...]
    w = w_ref[...]
    hT = lax.dot_general(w, xT, (((0,), (0,)), ((), ())),
                         preferred_element_type=jnp.float32)
    hT_ref[...] = hT
    a_s = jnp.sum(hT * asv_ref[...], axis=0, keepdims=True)
    a_d = jnp.sum(hT * adv_ref[...], axis=0, keepdims=True)
    as_ref[...] = a_s
    ad_ref[...] = a_d
    mx = jnp.max(a_s)
    mx_ref[...] = jnp.full((8, 128), mx, jnp.float32)
    m = jnp.maximum(a_d + mx, 0.0)
    z = a_s + a_d
    e = jnp.maximum(z, 0.2 * z)
    exs_ref[...] = jnp.exp(e - m)


_pre_call = pl.pallas_call(
    _pre_body,
    out_shape=[
        jax.ShapeDtypeStruct((D, NP), jnp.float32),
        jax.ShapeDtypeStruct((1, NP), jnp.float32),
        jax.ShapeDtypeStruct((1, NP), jnp.float32),
        jax.ShapeDtypeStruct((8, 128), jnp.float32),
        jax.ShapeDtypeStruct((1, NP), jnp.float32),
    ],
)


def _post_body(hT_ref, acc_ref, den_ref, exs_ref, b_ref, g_ref, be_ref, o_ref):
    exs = exs_ref[...]
    den = den_ref[...] + exs
    num = acc_ref[...] + exs * hT_ref[...]
    o = num / (den + 1e-16) + b_ref[...]
    mask = lax.broadcasted_iota(jnp.int32, (1, NP), 1) < N
    om = jnp.where(mask, o, 0.0)
    mu = jnp.sum(om, axis=1, keepdims=True) * (1.0 / N)
    dlt = jnp.where(mask, o - mu, 0.0)
    var = jnp.sum(dlt * dlt, axis=1, keepdims=True) * (1.0 / N)
    xn = g_ref[...] * (o - mu) * lax.rsqrt(var + 1e-5) + be_ref[...]
    o_ref[...] = jnp.where(mask, jnp.maximum(xn, 0.0), 0.0)


_post_call = pl.pallas_call(
    _post_body,
    out_shape=[jax.ShapeDtypeStruct((D, NP), jnp.float32)],
)


def _mlp_body(xT_ref, w1_ref, b1_ref, w2_ref, b2_ref, o_ref):
    xT = xT_ref[...]
    w1 = w1_ref[...]
    t = jnp.dot(w1, xT, preferred_element_type=jnp.float32) + b1_ref[...]
    t = jnp.maximum(t, 0.0)
    t = jnp.dot(w1, t, preferred_element_type=jnp.float32) + b1_ref[...]
    t = jnp.maximum(t, 0.0)
    o_ref[...] = (jnp.dot(w2_ref[...], t, preferred_element_type=jnp.float32)
                  + b2_ref[...])


_mlp_call = pl.pallas_call(
    _mlp_body,
    out_shape=[jax.ShapeDtypeStruct((D, NP), jnp.float32)],
)


def _sc_body(src_h, dst_h, as_h, ad_h, mx_h, hT_h,
             acc_h, den_h,
             as_v, ad_v, mx_v, h0, h1, h2, h3, a0, a1, a2, a3, den_v,
             srcb, dstb, n_edges):
    cid = lax.axis_index("c")
    sid = lax.axis_index("s")
    wid = sid * 2 + cid

    pltpu.sync_copy(as_h, as_v)
    pltpu.sync_copy(ad_h, ad_v)
    pltpu.sync_copy(mx_h, mx_v)
    hrows = (h0, h1, h2, h3)
    arows = (a0, a1, a2, a3)
    for k in range(FPT):
        pltpu.sync_copy(hT_h.at[wid * FPT + k], hrows[k])

    zeros = jnp.zeros((LANE,), jnp.float32)

    def zbody(i, _):
        for av in arows:
            av[pl.ds(i * LANE, LANE)] = zeros
        den_v[pl.ds(i * LANE, LANE)] = zeros
        return 0

    lax.fori_loop(0, NP // LANE, zbody, 0)

    mx16 = mx_v[...]
    nchunks = n_edges // CHUNK
    ngroups = CHUNK // LANE

    def chunk(c, _):
        pltpu.sync_copy(src_h.at[pl.ds(c * CHUNK, CHUNK)], srcb)
        pltpu.sync_copy(dst_h.at[pl.ds(c * CHUNK, CHUNK)], dstb)

        def grp(g, _):
            s16 = srcb[pl.ds(g * LANE, LANE)]
            d16 = dstb[pl.ds(g * LANE, LANE)]
            sA = plsc.load_gather(as_v, [s16])
            aD = plsc.load_gather(ad_v, [d16])
            z = sA + aD
            e = jnp.maximum(z, 0.2 * z)
            m = jnp.maximum(aD + mx16, 0.0)
            ex = jnp.exp(e - m)
            plsc.addupdate_scatter(den_v, [d16], ex)
            for hv, av in zip(hrows, arows):
                v = plsc.load_gather(hv, [s16]) * ex
                plsc.addupdate_scatter(av, [d16], v)
            return 0

        lax.fori_loop(0, ngroups, grp, 0)
        return 0

    lax.fori_loop(0, nchunks, chunk, 0)

    for k in range(FPT):
        pltpu.sync_copy(arows[k], acc_h.at[wid * FPT + k])

    @pl.when(wid == 0)
    def _():
        pltpu.sync_copy(den_v, den_h)


def _make_sc_call(n_edges):
    body = functools.partial(_sc_body, n_edges=n_edges)
    return pl.kernel(
        body,
        out_type=[
            jax.ShapeDtypeStruct((D, NP), jnp.float32),
            jax.ShapeDtypeStruct((NP,), jnp.float32),
        ],
        mesh=plsc.VectorSubcoreMesh(core_axis_name="c", subcore_axis_name="s"),
        compiler_params=pltpu.CompilerParams(needs_layout_passes=False),
        scratch_types=[
            pltpu.VMEM((NP,), jnp.float32),
            pltpu.VMEM((NP,), jnp.float32),
            pltpu.VMEM((LANE,), jnp.float32),
            pltpu.VMEM((NP,), jnp.float32),
            pltpu.VMEM((NP,), jnp.float32),
            pltpu.VMEM((NP,), jnp.float32),
            pltpu.VMEM((NP,), jnp.float32),
            pltpu.VMEM((NP,), jnp.float32),
            pltpu.VMEM((NP,), jnp.float32),
            pltpu.VMEM((NP,), jnp.float32),
            pltpu.VMEM((NP,), jnp.float32),
            pltpu.VMEM((NP,), jnp.float32),
            pltpu.VMEM((CHUNK,), jnp.int32),
            pltpu.VMEM((CHUNK,), jnp.int32),
        ],
    )


@jax.jit
def kernel(x, edge_index, params):
    src = edge_index[0]
    dst = edge_index[1]
    n_edges = src.shape[0]
    sc_call = _make_sc_call(n_edges)

    xT = jnp.pad(x.T, ((0, 0), (0, NP - x.shape[0])))
    for conv, bn in zip(params['convs'], params['bns']):
        asv = conv['att_src'].reshape(D, 1)
        adv = conv['att_dst'].reshape(D, 1)
        hT, a_s, a_d, mx, exs = _pre_call(xT, conv['W'], asv, adv)
        accT, den = sc_call(src, dst, a_s.reshape(NP), a_d.reshape(NP),
                            mx[0, :LANE], hT)
        xT = _post_call(hT, accT, den.reshape(1, NP), exs,
                        conv['bias'].reshape(D, 1),
                        bn['gamma'].reshape(D, 1),
                        bn['beta'].reshape(D, 1))[0]
    yT = _mlp_call(xT, params['W1'], params['b1'].reshape(D, 1),
                   params['W2'], params['b2'].reshape(D, 1))[0]
    return yT[:, :N].T
